# baseline (device time: 78636 ns/iter reference)
import jax
import jax.numpy as jnp
from jax import lax
from jax.experimental import pallas as pl
from jax.experimental.pallas import tpu as pltpu

N_DEV = 4
N_LAYERS = 3
N_SLOTS = 4


def kernel(x, Win0, Wout0, Win1, Wout1, Win2, Wout2):
    b, d_loc = x.shape
    _, h_dim = Win0.shape
    _, out_loc = Wout0.shape

    def body(x_ref, win0_ref, wout0_ref, win1_ref, wout1_ref, win2_ref,
             wout2_ref, out_ref, comm_ref, send_sems, recv_sems):
        my = lax.axis_index("i")
        left = (my + N_DEV - 1) % N_DEV
        right = (my + 1) % N_DEV

        barrier_sem = pltpu.get_barrier_semaphore()
        for nbr in (left, right):
            pl.semaphore_signal(
                barrier_sem, inc=1,
                device_id=(nbr,), device_id_type=pl.DeviceIdType.MESH,
            )
        pl.semaphore_wait(barrier_sem, 2)

        wins = (win0_ref, win1_ref, win2_ref)
        wouts = (wout0_ref, wout1_ref, wout2_ref)

        x_val = x_ref[:, :]
        hop = 0
        for k in range(N_LAYERS):
            partial = jnp.dot(
                x_val, wins[k][:, :], preferred_element_type=jnp.float32
            )
            comm_ref[hop % N_SLOTS] = partial
            acc = partial
            for _ in range(N_DEV - 1):
                s = hop % N_SLOTS
                r = (hop + 1) % N_SLOTS
                rdma = pltpu.make_async_remote_copy(
                    src_ref=comm_ref.at[s],
                    dst_ref=comm_ref.at[r],
                    send_sem=send_sems.at[s],
                    recv_sem=recv_sems.at[r],
                    device_id=(right,),
                    device_id_type=pl.DeviceIdType.MESH,
                )
                rdma.start()
                rdma.wait()
                acc = acc + comm_ref[r]
                hop += 1
            h = jnp.maximum(acc, 0.0)
            x_val = jnp.dot(
                h, wouts[k][:, :], preferred_element_type=jnp.float32
            )
        out_ref[:, :] = x_val

    return pl.pallas_call(
        body,
        out_shape=jax.ShapeDtypeStruct((b, out_loc), jnp.float32),
        in_specs=[pl.BlockSpec(memory_space=pltpu.VMEM)] * 7,
        out_specs=pl.BlockSpec(memory_space=pltpu.VMEM),
        scratch_shapes=[
            pltpu.VMEM((N_SLOTS, b, h_dim), jnp.float32),
            pltpu.SemaphoreType.DMA((N_SLOTS,)),
            pltpu.SemaphoreType.DMA((N_SLOTS,)),
        ],
        compiler_params=pltpu.CompilerParams(collective_id=0),
    )(x, Win0, Wout0, Win1, Wout1, Win2, Wout2)


# device time: 43586 ns/iter; 1.8042x vs baseline; 1.8042x over previous
import jax
import jax.numpy as jnp
from jax import lax
from jax.experimental import pallas as pl
from jax.experimental.pallas import tpu as pltpu

N_DEV = 4
N_LAYERS = 3
CHUNK = 128


def kernel(x, Win0, Wout0, Win1, Wout1, Win2, Wout2):
    b, d_loc = x.shape
    _, h_dim = Win0.shape
    _, out_loc = Wout0.shape

    def body(x_ref, win0_ref, wout0_ref, win1_ref, wout1_ref, win2_ref,
             wout2_ref, out_ref,
             p4_ref, rs_buf, ag_src, ag_buf,
             rs_sems, ag_sems, rs_send_sems, ag_send_sems, local_sem):
        my = lax.axis_index("i")

        barrier_sem = pltpu.get_barrier_semaphore()
        for j in range(1, N_DEV):
            pl.semaphore_signal(
                barrier_sem, inc=1,
                device_id=((my + j) % N_DEV,),
                device_id_type=pl.DeviceIdType.MESH,
            )
        pl.semaphore_wait(barrier_sem, N_DEV - 1)

        wins = (win0_ref, win1_ref, win2_ref)
        wouts = (wout0_ref, wout1_ref, wout2_ref)

        x_val = x_ref[:, :]
        for k in range(N_LAYERS):
            partial = jnp.dot(
                x_val, wins[k][:, :], preferred_element_type=jnp.float32
            )
            for j in range(N_DEV):
                p4_ref[j] = partial[:, j * CHUNK:(j + 1) * CHUNK]

            own_rs = pltpu.make_async_copy(
                p4_ref.at[my], rs_buf.at[my], local_sem
            )
            own_rs.start()
            rs_rdmas = []
            for j in range(1, N_DEV):
                t = (my + j) % N_DEV
                r = pltpu.make_async_remote_copy(
                    src_ref=p4_ref.at[t],
                    dst_ref=rs_buf.at[my],
                    send_sem=rs_send_sems.at[j - 1],
                    recv_sem=rs_sems.at[my],
                    device_id=(t,),
                    device_id_type=pl.DeviceIdType.MESH,
                )
                r.start()
                rs_rdmas.append(r)
            for j in range(1, N_DEV):
                s = (my + j) % N_DEV
                pltpu.make_async_remote_copy(
                    src_ref=p4_ref.at[s],
                    dst_ref=rs_buf.at[s],
                    send_sem=rs_send_sems.at[0],
                    recv_sem=rs_sems.at[s],
                    device_id=(s,),
                    device_id_type=pl.DeviceIdType.MESH,
                ).wait_recv()
            own_rs.wait()

            acc = rs_buf[0] + rs_buf[1] + rs_buf[2] + rs_buf[3]
            relu_c = jnp.maximum(acc, 0.0)
            ag_src[:, :] = relu_c

            own_ag = pltpu.make_async_copy(ag_src, ag_buf.at[my], local_sem)
            own_ag.start()
            ag_rdmas = []
            for j in range(1, N_DEV):
                t = (my + j) % N_DEV
                r = pltpu.make_async_remote_copy(
                    src_ref=ag_src,
                    dst_ref=ag_buf.at[my],
                    send_sem=ag_send_sems.at[j - 1],
                    recv_sem=ag_sems.at[my],
                    device_id=(t,),
                    device_id_type=pl.DeviceIdType.MESH,
                )
                r.start()
                ag_rdmas.append(r)
            for j in range(1, N_DEV):
                s = (my + j) % N_DEV
                pltpu.make_async_remote_copy(
                    src_ref=ag_src,
                    dst_ref=ag_buf.at[s],
                    send_sem=ag_send_sems.at[0],
                    recv_sem=ag_sems.at[s],
                    device_id=(s,),
                    device_id_type=pl.DeviceIdType.MESH,
                ).wait_recv()
            own_ag.wait()

            for r in rs_rdmas:
                r.wait_send()
            for r in ag_rdmas:
                r.wait_send()

            h = jnp.concatenate(
                [ag_buf[0], ag_buf[1], ag_buf[2], ag_buf[3]], axis=1
            )
            x_val = jnp.dot(
                h, wouts[k][:, :], preferred_element_type=jnp.float32
            )
        out_ref[:, :] = x_val

    return pl.pallas_call(
        body,
        out_shape=jax.ShapeDtypeStruct((b, out_loc), jnp.float32),
        in_specs=[pl.BlockSpec(memory_space=pltpu.VMEM)] * 7,
        out_specs=pl.BlockSpec(memory_space=pltpu.VMEM),
        scratch_shapes=[
            pltpu.VMEM((N_DEV, b, CHUNK), jnp.float32),
            pltpu.VMEM((N_DEV, b, CHUNK), jnp.float32),
            pltpu.VMEM((b, CHUNK), jnp.float32),
            pltpu.VMEM((N_DEV, b, CHUNK), jnp.float32),
            pltpu.SemaphoreType.DMA((N_DEV,)),
            pltpu.SemaphoreType.DMA((N_DEV,)),
            pltpu.SemaphoreType.DMA((N_DEV - 1,)),
            pltpu.SemaphoreType.DMA((N_DEV - 1,)),
            pltpu.SemaphoreType.DMA,
        ],
        compiler_params=pltpu.CompilerParams(collective_id=0),
    )(x, Win0, Wout0, Win1, Wout1, Win2, Wout2)


# device time: 34494 ns/iter; 2.2797x vs baseline; 1.2636x over previous
import jax
import jax.numpy as jnp
from jax import lax
from jax.experimental import pallas as pl
from jax.experimental.pallas import tpu as pltpu

N_DEV = 4
N_LAYERS = 3
CHUNK = 128


def kernel(x, Win0, Wout0, Win1, Wout1, Win2, Wout2):
    b, d_loc = x.shape
    _, h_dim = Win0.shape
    _, out_loc = Wout0.shape

    def body(x_ref, win0_ref, wout0_ref, win1_ref, wout1_ref, win2_ref,
             wout2_ref, out_ref,
             p4_ref, rs_buf, ag_src, ag_buf,
             rs_sems, ag_sems, rs_send_sems, ag_send_sems, local_sem):
        my = lax.axis_index("i")

        barrier_sem = pltpu.get_barrier_semaphore()
        for j in range(1, N_DEV):
            pl.semaphore_signal(
                barrier_sem, inc=1,
                device_id=((my + j) % N_DEV,),
                device_id_type=pl.DeviceIdType.MESH,
            )
        pl.semaphore_wait(barrier_sem, N_DEV - 1)

        wins = (win0_ref, win1_ref, win2_ref)
        wouts = (wout0_ref, wout1_ref, wout2_ref)

        x_val = x_ref[:, :]
        for k in range(N_LAYERS):
            partial = jnp.dot(
                x_val, wins[k][:, :], preferred_element_type=jnp.float32
            )
            partial_bf = partial.astype(jnp.bfloat16)
            for j in range(N_DEV):
                p4_ref[j] = partial_bf[:, j * CHUNK:(j + 1) * CHUNK]

            own_rs = pltpu.make_async_copy(
                p4_ref.at[my], rs_buf.at[my], local_sem
            )
            own_rs.start()
            rs_rdmas = []
            for j in range(1, N_DEV):
                t = (my + j) % N_DEV
                r = pltpu.make_async_remote_copy(
                    src_ref=p4_ref.at[t],
                    dst_ref=rs_buf.at[my],
                    send_sem=rs_send_sems.at[j - 1],
                    recv_sem=rs_sems.at[my],
                    device_id=(t,),
                    device_id_type=pl.DeviceIdType.MESH,
                )
                r.start()
                rs_rdmas.append(r)
            for j in range(1, N_DEV):
                s = (my + j) % N_DEV
                pltpu.make_async_remote_copy(
                    src_ref=p4_ref.at[s],
                    dst_ref=rs_buf.at[s],
                    send_sem=rs_send_sems.at[0],
                    recv_sem=rs_sems.at[s],
                    device_id=(s,),
                    device_id_type=pl.DeviceIdType.MESH,
                ).wait_recv()
            own_rs.wait()

            acc = (rs_buf[0].astype(jnp.float32)
                   + rs_buf[1].astype(jnp.float32)
                   + rs_buf[2].astype(jnp.float32)
                   + rs_buf[3].astype(jnp.float32))
            relu_c = jnp.maximum(acc, 0.0)
            ag_src[:, :] = relu_c.astype(jnp.bfloat16)

            own_ag = pltpu.make_async_copy(ag_src, ag_buf.at[my], local_sem)
            own_ag.start()
            ag_rdmas = []
            for j in range(1, N_DEV):
                t = (my + j) % N_DEV
                r = pltpu.make_async_remote_copy(
                    src_ref=ag_src,
                    dst_ref=ag_buf.at[my],
                    send_sem=ag_send_sems.at[j - 1],
                    recv_sem=ag_sems.at[my],
                    device_id=(t,),
                    device_id_type=pl.DeviceIdType.MESH,
                )
                r.start()
                ag_rdmas.append(r)
            for j in range(1, N_DEV):
                s = (my + j) % N_DEV
                pltpu.make_async_remote_copy(
                    src_ref=ag_src,
                    dst_ref=ag_buf.at[s],
                    send_sem=ag_send_sems.at[0],
                    recv_sem=ag_sems.at[s],
                    device_id=(s,),
                    device_id_type=pl.DeviceIdType.MESH,
                ).wait_recv()
            own_ag.wait()

            for r in rs_rdmas:
                r.wait_send()
            for r in ag_rdmas:
                r.wait_send()

            h = jnp.concatenate(
                [ag_buf[0], ag_buf[1], ag_buf[2], ag_buf[3]], axis=1
            ).astype(jnp.float32)
            x_val = jnp.dot(
                h, wouts[k][:, :], preferred_element_type=jnp.float32
            )
        out_ref[:, :] = x_val

    return pl.pallas_call(
        body,
        out_shape=jax.ShapeDtypeStruct((b, out_loc), jnp.float32),
        in_specs=[pl.BlockSpec(memory_space=pltpu.VMEM)] * 7,
        out_specs=pl.BlockSpec(memory_space=pltpu.VMEM),
        scratch_shapes=[
            pltpu.VMEM((N_DEV, b, CHUNK), jnp.bfloat16),
            pltpu.VMEM((N_DEV, b, CHUNK), jnp.bfloat16),
            pltpu.VMEM((b, CHUNK), jnp.bfloat16),
            pltpu.VMEM((N_DEV, b, CHUNK), jnp.bfloat16),
            pltpu.SemaphoreType.DMA((N_DEV,)),
            pltpu.SemaphoreType.DMA((N_DEV,)),
            pltpu.SemaphoreType.DMA((N_DEV - 1,)),
            pltpu.SemaphoreType.DMA((N_DEV - 1,)),
            pltpu.SemaphoreType.DMA,
        ],
        compiler_params=pltpu.CompilerParams(collective_id=0),
    )(x, Win0, Wout0, Win1, Wout1, Win2, Wout2)
